# baseline (speedup 1.0000x reference)
import jax
import jax.numpy as jnp
from jax.experimental import pallas as pl
from jax.experimental.pallas import tpu as pltpu

def _copy_body(in_ref, out_ref):
    out_ref[...] = in_ref[...]


def kernel(pred):
    b, c, h, w = pred.shape
    return pl.pallas_call(
        _copy_body,
        out_shape=jax.ShapeDtypeStruct(pred.shape, pred.dtype),
        grid=(2 * b,),
        in_specs=[pl.BlockSpec((1, c // 2, h, w), lambda i: (i // 2, i % 2, 0, 0))],
        out_specs=pl.BlockSpec((1, c // 2, h, w), lambda i: (i // 2, i % 2, 0, 0)),
        compiler_params=pltpu.CompilerParams(
            dimension_semantics=("parallel",),
        ),
    )(pred)


# speedup vs baseline: 1.1274x; 1.1274x over previous
import jax
import jax.numpy as jnp
from jax.experimental import pallas as pl
from jax.experimental.pallas import tpu as pltpu

def _copy_body(in_ref, out_ref):
    out_ref[...] = in_ref[...]


def kernel(pred):
    b, c, h, w = pred.shape
    return pl.pallas_call(
        _copy_body,
        out_shape=jax.ShapeDtypeStruct(pred.shape, pred.dtype),
        grid=(b // 2,),
        in_specs=[pl.BlockSpec((2, c, h, w), lambda i: (i, 0, 0, 0))],
        out_specs=pl.BlockSpec((2, c, h, w), lambda i: (i, 0, 0, 0)),
        compiler_params=pltpu.CompilerParams(
            dimension_semantics=("parallel",),
        ),
    )(pred)
